# Initial kernel scaffold; baseline (speedup 1.0000x reference)
import jax
import jax.numpy as jnp
from jax.experimental import pallas as pl


def kernel(x, A_indices, A_values, W1, b1, W2, b2):
    raise NotImplementedError("write your pallas kernel here")


# speedup vs baseline: 3.0750x; 3.0750x over previous
import jax
import jax.numpy as jnp
from jax import lax
from jax.experimental import pallas as pl
from jax.experimental.pallas import tpu as pltpu
from jax.experimental.pallas import tpu_sc as plsc

N_NODES = 10000
N_EDGES = 320000
D = 128
DH = D // 2

NC = 2
NS = 16
EPT = N_EDGES // NS
K = 80
CHUNKS = EPT // K
ROWS_PER_TILE = 640
NPAD = NS * ROWS_PER_TILE


def _mm_bias_kernel(x_ref, wt_ref, b_ref, o_ref):
    o_ref[...] = (
        jnp.dot(x_ref[...], wt_ref[...], preferred_element_type=jnp.float32)
        + b_ref[...]
    )


def _mm_bias(x, wt, b):
    blk = 1000
    return pl.pallas_call(
        _mm_bias_kernel,
        grid=(N_NODES // blk,),
        in_specs=[
            pl.BlockSpec((blk, D), lambda i: (i, 0)),
            pl.BlockSpec((D, D), lambda i: (0, 0)),
            pl.BlockSpec((1, D), lambda i: (0, 0)),
        ],
        out_specs=pl.BlockSpec((blk, D), lambda i: (i, 0)),
        out_shape=jax.ShapeDtypeStruct((N_NODES, D), jnp.float32),
    )(x, wt, b)


def _fused_relu_mm_kernel(p0_ref, p1_ref, wta_ref, wtb_ref, b_ref, o_ref):
    acc = jnp.dot(
        jax.nn.relu(p0_ref[...]), wta_ref[...],
        preferred_element_type=jnp.float32,
    )
    acc += jnp.dot(
        jax.nn.relu(p1_ref[...]), wtb_ref[...],
        preferred_element_type=jnp.float32,
    )
    o_ref[...] = acc + b_ref[...]


def _fused_relu_mm(p0, p1, wta, wtb, b):
    blk = 1000
    return pl.pallas_call(
        _fused_relu_mm_kernel,
        grid=(N_NODES // blk,),
        in_specs=[
            pl.BlockSpec((blk, DH), lambda i: (i, 0)),
            pl.BlockSpec((blk, DH), lambda i: (i, 0)),
            pl.BlockSpec((DH, D), lambda i: (0, 0)),
            pl.BlockSpec((DH, D), lambda i: (0, 0)),
            pl.BlockSpec((1, D), lambda i: (0, 0)),
        ],
        out_specs=pl.BlockSpec((blk, D), lambda i: (i, 0)),
        out_shape=jax.ShapeDtypeStruct((N_NODES, D), jnp.float32),
    )(p0, p1, wta, wtb, b)


def _bcast16(vec, e):
    idx = jnp.full((16, 1), e, jnp.int32)
    dnums = lax.GatherDimensionNumbers(
        offset_dims=(), collapsed_slice_dims=(0,), start_index_map=(0,)
    )
    return lax.gather(
        vec, idx, dnums, (1,), mode=lax.GatherScatterMode.PROMISE_IN_BOUNDS
    )


def _spmm_body(h_hbm, edata_hbm, vals_hbm, zeros_hbm, out_hbm,
               ebuf, vbuf, cidx_v, gbuf, acc, sem):
    cid = lax.axis_index("c")
    sid = lax.axis_index("s")

    pltpu.sync_copy(zeros_hbm, acc.at[pl.ds(sid * ROWS_PER_TILE, ROWS_PER_TILE)])
    plsc.subcore_barrier()

    cid_vec = jnp.full((16,), cid, jnp.int32)

    def chunk_body(i, carry):
        pltpu.sync_copy(edata_hbm.at[sid, i], ebuf)
        pltpu.sync_copy(vals_hbm.at[sid, i], vbuf)
        for g in range(K // 16):
            sl = pl.ds(16 * g, 16)
            cidx_v[sl] = ebuf[0, sl] * 2 + cid_vec
        pltpu.async_copy(h_hbm.at[cidx_v], gbuf, sem).wait()
        for g in range(K // 16):
            vv = vbuf[pl.ds(16 * g, 16)]
            for e in range(16):
                vb = _bcast16(vv, e)
                r = 16 * g + e
                for j in range(DH // 16):
                    sl = pl.ds(16 * j, 16)
                    gbuf[r, sl] = gbuf[r, sl] * vb
        pltpu.sync_copy(gbuf, acc.at[ebuf.at[1]], add=True)
        return carry

    lax.fori_loop(0, CHUNKS, chunk_body, 0)

    plsc.subcore_barrier()

    sl = pl.ds(sid * ROWS_PER_TILE, ROWS_PER_TILE)
    pltpu.sync_copy(acc.at[sl], out_hbm.at[cid].at[sl])


_spmm = pl.kernel(
    _spmm_body,
    out_type=jax.ShapeDtypeStruct((NC, NPAD, DH), jnp.float32),
    mesh=plsc.VectorSubcoreMesh(core_axis_name="c", subcore_axis_name="s"),
    compiler_params=pltpu.CompilerParams(use_tc_tiling_on_sc=False),
    scratch_types=[
        pltpu.VMEM((2, K), jnp.int32),
        pltpu.VMEM((K,), jnp.float32),
        pltpu.VMEM((K,), jnp.int32),
        pltpu.VMEM((K, DH), jnp.float32),
        pltpu.VMEM_SHARED((NPAD, DH), jnp.float32),
        pltpu.SemaphoreType.DMA,
    ],
)


def kernel(x, A_indices, A_values, W1, b1, W2, b2):
    cols = A_indices[1].astype(jnp.int32).reshape(NS, CHUNKS, K)
    rows = A_indices[0].astype(jnp.int32).reshape(NS, CHUNKS, K)
    vals = A_values.reshape(NS, CHUNKS, K)
    edata = jnp.stack([cols, rows], axis=2)
    zeros = jnp.zeros((ROWS_PER_TILE, DH), jnp.float32)

    h1 = _mm_bias(x, W1.T, b1.reshape(1, D))
    p = _spmm(h1.reshape(2 * N_NODES, DH), edata, vals, zeros)
    h2 = _fused_relu_mm(
        p[0, :N_NODES], p[1, :N_NODES], W2.T[:DH], W2.T[DH:], b2.reshape(1, D)
    )
    q = _spmm(h2.reshape(2 * N_NODES, DH), edata, vals, zeros)
    return jnp.concatenate([q[0, :N_NODES], q[1, :N_NODES]], axis=1)
